# baseline (speedup 1.0000x reference)
import functools

import jax
import jax.numpy as jnp
from jax import lax
from jax.experimental import pallas as pl
from jax.experimental.pallas import tpu as pltpu

N_NODES_ = 10000
D_ = 256
E_ = 160000
INV_T = 10.0


def _norm_body(z_ref, zn_ref):
    x = z_ref[...]
    n2 = jnp.sum(x * x, axis=1, keepdims=True)
    inv = lax.rsqrt(jnp.maximum(n2, 1e-16))
    zn_ref[...] = x * inv


def _normalize(z):
    return pl.pallas_call(
        _norm_body,
        out_shape=jax.ShapeDtypeStruct((N_NODES_, D_), jnp.float32),
    )(z)


def _sims_body(pa_ref, pb_ref, na_ref, nb_ref, ps_ref, ns_ref):
    ps_ref[...] = jnp.sum(pa_ref[...] * pb_ref[...], axis=1, keepdims=True) * INV_T
    ns_ref[...] = jnp.sum(na_ref[...] * nb_ref[...], axis=1, keepdims=True) * INV_T


def _sims(pa, pb, na, nb):
    blk = 2000
    grid = E_ // blk
    return pl.pallas_call(
        _sims_body,
        grid=(grid,),
        in_specs=[pl.BlockSpec((blk, D_), lambda i: (i, 0))] * 4,
        out_specs=[pl.BlockSpec((blk, 1), lambda i: (i, 0))] * 2,
        out_shape=[jax.ShapeDtypeStruct((E_, 1), jnp.float32)] * 2,
    )(pa, pb, na, nb)


def _loss_body(ps_ref, ns_ref, out_ref, *, n_valid):
    rows, cols = ps_ref.shape
    ridx = lax.broadcasted_iota(jnp.int32, (rows, cols), 0)
    cidx = lax.broadcasted_iota(jnp.int32, (rows, cols), 1)
    valid = (ridx * cols + cidx) < n_valid
    ns = ns_ref[...]
    nsum = jnp.sum(jnp.where(valid, jnp.exp(ns), 0.0))
    ps = ps_ref[...]
    loss = jnp.where(valid, jnp.log(jnp.exp(ps) + nsum) - ps, 0.0)
    out_ref[...] = (jnp.sum(loss) / n_valid).reshape(1, 1)


def _loss(ps2d, ns2d, n_valid):
    out = pl.pallas_call(
        functools.partial(_loss_body, n_valid=n_valid),
        out_shape=jax.ShapeDtypeStruct((1, 1), jnp.float32),
    )(ps2d, ns2d)
    return out.reshape(())


def kernel(z, edge_index, negative_edge_index):
    zn = _normalize(z)
    pa = jnp.take(zn, edge_index[0], axis=0)
    pb = jnp.take(zn, edge_index[1], axis=0)
    na = jnp.take(zn, negative_edge_index[0], axis=0)
    nb = jnp.take(zn, negative_edge_index[1], axis=0)
    ps, ns = _sims(pa, pb, na, nb)
    return _loss(ps.reshape(E_ // 128, 128), ns.reshape(E_ // 128, 128), E_)
